# baseline (speedup 1.0000x reference)
import functools

import jax
import jax.numpy as jnp
from jax import lax
from jax.experimental import pallas as pl
from jax.experimental.pallas import tpu as pltpu
from jax.experimental.pallas import tpu_sc as plsc

EMB = 64
VOCAB = 5
PAD_IDX = VOCAB - 1
BATCH = 4096
SEQ = 200

BB = 128


EMB_PAD = 128


def _sc_species_gather(cat_emb_pad, species):
    info = plsc.get_sparse_core_info()
    nc, ns = info.num_cores, info.num_subcores
    nw = nc * ns
    b_per_w = BATCH // nw

    mesh = plsc.VectorSubcoreMesh(core_axis_name="c", subcore_axis_name="s")

    @functools.partial(
        pl.kernel,
        mesh=mesh,
        out_type=jax.ShapeDtypeStruct((BATCH, EMB_PAD), jnp.float32),
        scratch_types=[
            pltpu.VMEM((b_per_w,), jnp.int32),
            pltpu.VMEM((b_per_w, EMB_PAD), jnp.float32),
            pltpu.SemaphoreType.DMA,
        ],
    )
    def gather_kernel(table_hbm, idx_hbm, out_hbm, idx_v, rows_v, sem):
        wid = lax.axis_index("s") * nc + lax.axis_index("c")
        base = wid * b_per_w
        pltpu.sync_copy(idx_hbm.at[pl.ds(base, b_per_w)], idx_v)
        pltpu.async_copy(table_hbm.at[idx_v], rows_v, sem).wait()
        pltpu.sync_copy(rows_v, out_hbm.at[pl.ds(base, b_per_w)])

    return gather_kernel(cat_emb_pad, species)


def _tc_body(seqs_ref, spemb_ref, vocab_ref, total_ref, spout_ref):
    s = seqs_ref[...]
    spe = spemb_ref[:, :EMB]
    vt = vocab_ref[...]
    row = lax.broadcasted_iota(jnp.int32, (VOCAB, 1), 0)
    vt = jnp.where(row == PAD_IDX, 0.0, vt)
    spout_ref[...] = spe
    idx = jnp.broadcast_to(s[:, :, None], (BB, SEQ, EMB))
    xt = jnp.broadcast_to(vt[None], (BB, VOCAB, EMB))
    seq_emb = jnp.take_along_axis(xt, idx, axis=1, mode="promise_in_bounds")
    total_ref[...] = seq_emb + spe[:, None, :]


def _tc_fused(seqs, spemb_pad, vocab_emb):
    nb = BATCH // BB
    return pl.pallas_call(
        _tc_body,
        grid=(nb,),
        in_specs=[
            pl.BlockSpec((BB, SEQ), lambda i: (i, 0)),
            pl.BlockSpec((BB, EMB_PAD), lambda i: (i, 0)),
            pl.BlockSpec((VOCAB, EMB), lambda i: (0, 0)),
        ],
        out_specs=[
            pl.BlockSpec((BB, SEQ, EMB), lambda i: (i, 0, 0)),
            pl.BlockSpec((BB, EMB), lambda i: (i, 0)),
        ],
        out_shape=[
            jax.ShapeDtypeStruct((BATCH, SEQ, EMB), jnp.float32),
            jax.ShapeDtypeStruct((BATCH, EMB), jnp.float32),
        ],
        compiler_params=pltpu.CompilerParams(
            dimension_semantics=("parallel",),
        ),
    )(seqs, spemb_pad, vocab_emb)


def kernel(seqs, species, vocab_emb, cat_emb):
    seqs = seqs.astype(jnp.int32)
    species = species.astype(jnp.int32)
    cat_emb_pad = jnp.pad(cat_emb, ((0, 0), (0, EMB_PAD - EMB)))
    spemb_pad = _sc_species_gather(cat_emb_pad, species)
    total, spemb = _tc_fused(seqs, spemb_pad, vocab_emb)
    return total, spemb


# speedup vs baseline: 1.1144x; 1.1144x over previous
import functools

import jax
import jax.numpy as jnp
from jax import lax
from jax.experimental import pallas as pl
from jax.experimental.pallas import tpu as pltpu
from jax.experimental.pallas import tpu_sc as plsc

EMB = 64
VOCAB = 5
PAD_IDX = VOCAB - 1
BATCH = 4096
SEQ = 200

BB = 128


EMB_PAD = 128


def _sc_species_gather(cat_emb_pad, species):
    info = plsc.get_sparse_core_info()
    nc, ns = info.num_cores, info.num_subcores
    nw = nc * ns
    b_per_w = BATCH // nw

    mesh = plsc.VectorSubcoreMesh(core_axis_name="c", subcore_axis_name="s")

    @functools.partial(
        pl.kernel,
        mesh=mesh,
        out_type=jax.ShapeDtypeStruct((BATCH, EMB_PAD), jnp.float32),
        scratch_types=[
            pltpu.VMEM((b_per_w,), jnp.int32),
            pltpu.VMEM((b_per_w, EMB_PAD), jnp.float32),
            pltpu.SemaphoreType.DMA,
        ],
    )
    def gather_kernel(table_hbm, idx_hbm, out_hbm, idx_v, rows_v, sem):
        wid = lax.axis_index("s") * nc + lax.axis_index("c")
        base = wid * b_per_w
        pltpu.sync_copy(idx_hbm.at[pl.ds(base, b_per_w)], idx_v)
        pltpu.async_copy(table_hbm.at[idx_v], rows_v, sem).wait()
        pltpu.sync_copy(rows_v, out_hbm.at[pl.ds(base, b_per_w)])

    return gather_kernel(cat_emb_pad, species)


def _tc_body(se_ref, so_ref, spemb_ref, vocab_ref, total_ref, spout_ref):
    HS = SEQ // 2
    se = se_ref[...]
    so = so_ref[...]
    spe = spemb_ref[:, :EMB]
    vt = vocab_ref[...]
    row = lax.broadcasted_iota(jnp.int32, (VOCAB, 1), 0)
    vt = jnp.where(row == PAD_IDX, 0.0, vt)
    spout_ref[...] = spe
    eb = jnp.broadcast_to(se[:, :, None], (BB, HS, 2 * EMB))
    ob = jnp.broadcast_to(so[:, :, None], (BB, HS, 2 * EMB))
    lane = lax.broadcasted_iota(jnp.int32, (BB, HS, 2 * EMB), 2)
    idx2 = jnp.where(lane < EMB, eb, ob)
    vt2 = jnp.concatenate([vt, vt], axis=-1)
    xt = jnp.broadcast_to(vt2[None], (BB, VOCAB, 2 * EMB))
    seq_emb = jnp.take_along_axis(xt, idx2, axis=1, mode="promise_in_bounds")
    spe2 = jnp.concatenate([spe, spe], axis=-1)
    total_ref[...] = seq_emb + spe2[:, None, :]


def _tc_fused(seqs, spemb_pad, vocab_emb):
    nb = BATCH // BB
    total2, spemb = pl.pallas_call(
        _tc_body,
        grid=(nb,),
        in_specs=[
            pl.BlockSpec((BB, SEQ // 2), lambda i: (i, 0)),
            pl.BlockSpec((BB, SEQ // 2), lambda i: (i, 0)),
            pl.BlockSpec((BB, EMB_PAD), lambda i: (i, 0)),
            pl.BlockSpec((VOCAB, EMB), lambda i: (0, 0)),
        ],
        out_specs=[
            pl.BlockSpec((BB, SEQ // 2, 2 * EMB), lambda i: (i, 0, 0)),
            pl.BlockSpec((BB, EMB), lambda i: (i, 0)),
        ],
        out_shape=[
            jax.ShapeDtypeStruct((BATCH, SEQ // 2, 2 * EMB), jnp.float32),
            jax.ShapeDtypeStruct((BATCH, EMB), jnp.float32),
        ],
        compiler_params=pltpu.CompilerParams(
            dimension_semantics=("parallel",),
        ),
    )(seqs[:, 0::2], seqs[:, 1::2], spemb_pad, vocab_emb)
    return total2.reshape(BATCH, SEQ, EMB), spemb


def kernel(seqs, species, vocab_emb, cat_emb):
    seqs = seqs.astype(jnp.int32)
    species = species.astype(jnp.int32)
    cat_emb_pad = jnp.pad(cat_emb, ((0, 0), (0, EMB_PAD - EMB)))
    spemb_pad = _sc_species_gather(cat_emb_pad, species)
    return _tc_fused(seqs, spemb_pad, vocab_emb)


# speedup vs baseline: 1.3077x; 1.1734x over previous
import functools

import jax
import jax.numpy as jnp
from jax import lax
from jax.experimental import pallas as pl
from jax.experimental.pallas import tpu as pltpu
from jax.experimental.pallas import tpu_sc as plsc

EMB = 64
VOCAB = 5
PAD_IDX = VOCAB - 1
BATCH = 4096
SEQ = 200

BB = 128


EMB_PAD = 128


def _sc_species_gather(cat_emb_pad, species):
    info = plsc.get_sparse_core_info()
    nc, ns = info.num_cores, info.num_subcores
    nw = nc * ns
    b_per_w = BATCH // nw

    mesh = plsc.VectorSubcoreMesh(core_axis_name="c", subcore_axis_name="s")

    @functools.partial(
        pl.kernel,
        mesh=mesh,
        out_type=jax.ShapeDtypeStruct((BATCH, EMB_PAD), jnp.float32),
        scratch_types=[
            pltpu.VMEM((b_per_w,), jnp.int32),
            pltpu.VMEM((b_per_w, EMB_PAD), jnp.float32),
            pltpu.SemaphoreType.DMA,
        ],
    )
    def gather_kernel(table_hbm, idx_hbm, out_hbm, idx_v, rows_v, sem):
        wid = lax.axis_index("s") * nc + lax.axis_index("c")
        base = wid * b_per_w
        pltpu.sync_copy(idx_hbm.at[pl.ds(base, b_per_w)], idx_v)
        pltpu.async_copy(table_hbm.at[idx_v], rows_v, sem).wait()
        pltpu.sync_copy(rows_v, out_hbm.at[pl.ds(base, b_per_w)])

    return gather_kernel(cat_emb_pad, species)


def _tc_body(sw_ref, spemb_ref, vocab_ref, total_ref, spout_ref):
    HS = SEQ // 2
    sw = sw_ref[...]
    spe = spemb_ref[:, :EMB]
    vt = vocab_ref[...]
    row = lax.broadcasted_iota(jnp.int32, (VOCAB, 1), 0)
    vt = jnp.where(row == PAD_IDX, 0.0, vt)
    spout_ref[...] = spe
    swb = jnp.broadcast_to(sw[:, :, None], (BB, HS, 2 * EMB))
    lane = lax.broadcasted_iota(jnp.int32, (BB, HS, 2 * EMB), 2)
    idx2 = jnp.where(lane < EMB, swb & 7, swb >> 3)
    vt2 = jnp.concatenate([vt, vt], axis=-1)
    xt = jnp.broadcast_to(vt2[None], (BB, VOCAB, 2 * EMB))
    seq_emb = jnp.take_along_axis(xt, idx2, axis=1, mode="promise_in_bounds")
    spe2 = jnp.concatenate([spe, spe], axis=-1)
    total_ref[...] = seq_emb + spe2[:, None, :]


def _tc_fused(seqs, spemb_pad, vocab_emb):
    nb = BATCH // BB
    sw = seqs[:, 0::2] + (seqs[:, 1::2] << 3)
    total2, spemb = pl.pallas_call(
        _tc_body,
        grid=(nb,),
        in_specs=[
            pl.BlockSpec((BB, SEQ // 2), lambda i: (i, 0)),
            pl.BlockSpec((BB, EMB_PAD), lambda i: (i, 0)),
            pl.BlockSpec((VOCAB, EMB), lambda i: (0, 0)),
        ],
        out_specs=[
            pl.BlockSpec((BB, SEQ // 2, 2 * EMB), lambda i: (i, 0, 0)),
            pl.BlockSpec((BB, EMB), lambda i: (i, 0)),
        ],
        out_shape=[
            jax.ShapeDtypeStruct((BATCH, SEQ // 2, 2 * EMB), jnp.float32),
            jax.ShapeDtypeStruct((BATCH, EMB), jnp.float32),
        ],
        compiler_params=pltpu.CompilerParams(
            dimension_semantics=("parallel",),
        ),
    )(sw, spemb_pad, vocab_emb)
    return total2.reshape(BATCH, SEQ, EMB), spemb


def kernel(seqs, species, vocab_emb, cat_emb):
    seqs = seqs.astype(jnp.int32)
    species = species.astype(jnp.int32)
    cat_emb_pad = jnp.pad(cat_emb, ((0, 0), (0, EMB_PAD - EMB)))
    spemb_pad = _sc_species_gather(cat_emb_pad, species)
    return _tc_fused(seqs, spemb_pad, vocab_emb)


# speedup vs baseline: 1.4328x; 1.0957x over previous
import functools

import jax
import jax.numpy as jnp
from jax import lax
from jax.experimental import pallas as pl
from jax.experimental.pallas import tpu as pltpu
from jax.experimental.pallas import tpu_sc as plsc

EMB = 64
VOCAB = 5
PAD_IDX = VOCAB - 1
BATCH = 4096
SEQ = 200

BB = 128


EMB_PAD = 128


def _sc_species_gather(cat_emb_pad, species):
    info = plsc.get_sparse_core_info()
    nc, ns = info.num_cores, info.num_subcores
    nw = nc * ns
    b_per_w = BATCH // nw

    mesh = plsc.VectorSubcoreMesh(core_axis_name="c", subcore_axis_name="s")

    @functools.partial(
        pl.kernel,
        mesh=mesh,
        out_type=jax.ShapeDtypeStruct((BATCH, EMB_PAD), jnp.float32),
        scratch_types=[
            pltpu.VMEM((b_per_w,), jnp.int32),
            pltpu.VMEM((b_per_w, EMB_PAD), jnp.float32),
            pltpu.SemaphoreType.DMA,
        ],
    )
    def gather_kernel(table_hbm, idx_hbm, out_hbm, idx_v, rows_v, sem):
        wid = lax.axis_index("s") * nc + lax.axis_index("c")
        base = wid * b_per_w
        pltpu.sync_copy(idx_hbm.at[pl.ds(base, b_per_w)], idx_v)
        pltpu.async_copy(table_hbm.at[idx_v], rows_v, sem).wait()
        pltpu.sync_copy(rows_v, out_hbm.at[pl.ds(base, b_per_w)])

    return gather_kernel(cat_emb_pad, species)


def _tc_body(sw_ref, spemb_ref, vocab2_ref, total_ref, spout_ref):
    HS = SEQ // 2
    sw = sw_ref[...]
    spe2 = spemb_ref[...]
    vt2 = vocab2_ref[...]
    spout_ref[...] = spe2[:, :EMB]
    swb = jnp.broadcast_to(sw[:, :, None], (BB, HS, 2 * EMB))
    lane = lax.broadcasted_iota(jnp.int32, (BB, HS, 2 * EMB), 2)
    sh = jnp.where(lane < EMB, 0, 3)
    idx2 = (swb >> sh) & 7
    xt = jnp.broadcast_to(vt2[None], (BB, VOCAB, 2 * EMB))
    seq_emb = jnp.take_along_axis(xt, idx2, axis=1, mode="promise_in_bounds")
    total_ref[...] = seq_emb + spe2[:, None, :]


def _tc_fused(seqs, spemb2, vocab2):
    nb = BATCH // BB
    sw = seqs[:, 0::2] + (seqs[:, 1::2] << 3)
    total2, spemb = pl.pallas_call(
        _tc_body,
        grid=(nb,),
        in_specs=[
            pl.BlockSpec((BB, SEQ // 2), lambda i: (i, 0)),
            pl.BlockSpec((BB, 2 * EMB), lambda i: (i, 0)),
            pl.BlockSpec((VOCAB, 2 * EMB), lambda i: (0, 0)),
        ],
        out_specs=[
            pl.BlockSpec((BB, SEQ // 2, 2 * EMB), lambda i: (i, 0, 0)),
            pl.BlockSpec((BB, EMB), lambda i: (i, 0)),
        ],
        out_shape=[
            jax.ShapeDtypeStruct((BATCH, SEQ // 2, 2 * EMB), jnp.float32),
            jax.ShapeDtypeStruct((BATCH, EMB), jnp.float32),
        ],
        compiler_params=pltpu.CompilerParams(
            dimension_semantics=("parallel",),
        ),
    )(sw, spemb2, vocab2)
    return total2.reshape(BATCH, SEQ, EMB), spemb


def kernel(seqs, species, vocab_emb, cat_emb):
    seqs = seqs.astype(jnp.int32)
    species = species.astype(jnp.int32)
    cat_emb2 = jnp.concatenate([cat_emb, cat_emb], axis=1)
    vt = vocab_emb.at[PAD_IDX].set(0.0)
    vocab2 = jnp.concatenate([vt, vt], axis=1)
    spemb2 = _sc_species_gather(cat_emb2, species)
    return _tc_fused(seqs, spemb2, vocab2)


# speedup vs baseline: 1.4441x; 1.0079x over previous
import functools

import jax
import jax.numpy as jnp
from jax import lax
from jax.experimental import pallas as pl
from jax.experimental.pallas import tpu as pltpu
from jax.experimental.pallas import tpu_sc as plsc

EMB = 64
VOCAB = 5
PAD_IDX = VOCAB - 1
BATCH = 4096
SEQ = 200

BB = 256


EMB_PAD = 128


def _sc_species_gather(cat_emb_pad, species):
    info = plsc.get_sparse_core_info()
    nc, ns = info.num_cores, info.num_subcores
    nw = nc * ns
    b_per_w = BATCH // nw

    mesh = plsc.VectorSubcoreMesh(core_axis_name="c", subcore_axis_name="s")

    @functools.partial(
        pl.kernel,
        mesh=mesh,
        out_type=jax.ShapeDtypeStruct((BATCH, EMB_PAD), jnp.float32),
        scratch_types=[
            pltpu.VMEM((b_per_w,), jnp.int32),
            pltpu.VMEM((b_per_w, EMB_PAD), jnp.float32),
            pltpu.SemaphoreType.DMA,
        ],
    )
    def gather_kernel(table_hbm, idx_hbm, out_hbm, idx_v, rows_v, sem):
        wid = lax.axis_index("s") * nc + lax.axis_index("c")
        base = wid * b_per_w
        pltpu.sync_copy(idx_hbm.at[pl.ds(base, b_per_w)], idx_v)
        pltpu.async_copy(table_hbm.at[idx_v], rows_v, sem).wait()
        pltpu.sync_copy(rows_v, out_hbm.at[pl.ds(base, b_per_w)])

    return gather_kernel(cat_emb_pad, species)


def _tc_body(sw_ref, spemb_ref, vocab2_ref, total_ref, spout_ref):
    HS = SEQ // 2
    sw = sw_ref[...]
    spe2 = spemb_ref[...]
    vt2 = vocab2_ref[...]
    spout_ref[...] = spe2[:, :EMB]
    swb = jnp.broadcast_to(sw[:, :, None], (BB, HS, 2 * EMB))
    lane = lax.broadcasted_iota(jnp.int32, (BB, HS, 2 * EMB), 2)
    sh = jnp.where(lane < EMB, 0, 3)
    idx2 = (swb >> sh) & 7
    xt = jnp.broadcast_to(vt2[None], (BB, VOCAB, 2 * EMB))
    seq_emb = jnp.take_along_axis(xt, idx2, axis=1, mode="promise_in_bounds")
    total_ref[...] = seq_emb + spe2[:, None, :]


def _tc_fused(seqs, spemb2, vocab2):
    nb = BATCH // BB
    sw = seqs[:, 0::2] + (seqs[:, 1::2] << 3)
    total2, spemb = pl.pallas_call(
        _tc_body,
        grid=(nb,),
        in_specs=[
            pl.BlockSpec((BB, SEQ // 2), lambda i: (i, 0)),
            pl.BlockSpec((BB, 2 * EMB), lambda i: (i, 0)),
            pl.BlockSpec((VOCAB, 2 * EMB), lambda i: (0, 0)),
        ],
        out_specs=[
            pl.BlockSpec((BB, SEQ // 2, 2 * EMB), lambda i: (i, 0, 0)),
            pl.BlockSpec((BB, EMB), lambda i: (i, 0)),
        ],
        out_shape=[
            jax.ShapeDtypeStruct((BATCH, SEQ // 2, 2 * EMB), jnp.float32),
            jax.ShapeDtypeStruct((BATCH, EMB), jnp.float32),
        ],
        compiler_params=pltpu.CompilerParams(
            dimension_semantics=("parallel",),
        ),
    )(sw, spemb2, vocab2)
    return total2.reshape(BATCH, SEQ, EMB), spemb


def kernel(seqs, species, vocab_emb, cat_emb):
    seqs = seqs.astype(jnp.int32)
    species = species.astype(jnp.int32)
    cat_emb2 = jnp.concatenate([cat_emb, cat_emb], axis=1)
    vt = vocab_emb.at[PAD_IDX].set(0.0)
    vocab2 = jnp.concatenate([vt, vt], axis=1)
    spemb2 = _sc_species_gather(cat_emb2, species)
    return _tc_fused(seqs, spemb2, vocab2)
